# baseline (speedup 1.0000x reference)
import functools

import jax
import jax.numpy as jnp
from jax import lax
from jax.experimental import pallas as pl
from jax.experimental.pallas import tpu as pltpu
from jax.experimental.pallas import tpu_sc as plsc

N = 10000
NP = 10240
H = 128
E_RAW = 320000
E_SELF = E_RAW + N
NC = 2
NS = 16
NW = NC * NS
C = 128
K = 82
E_PAD = NW * C * K
ROWS_PER_TILE = NP // NS
_ZCHUNKS = (128,) * (ROWS_PER_TILE // 128)


def _proj_body(x_ref, w_ref, asv_ref, adv_ref, h_ref, as_ref, ad_ref, sh_ref):
    h = jnp.dot(x_ref[...], w_ref[...], preferred_element_type=jnp.float32)
    h_ref[...] = h
    a_s = jnp.sum(h * asv_ref[...], axis=1, keepdims=True)
    a_d = jnp.sum(h * adv_ref[...], axis=1, keepdims=True)
    valid = jax.lax.broadcasted_iota(jnp.int32, (NP, 1), 0) < N
    a_s = jnp.where(valid, a_s, -1e30)
    a_d = jnp.where(valid, a_d, -1e30)
    as_ref[...] = a_s
    ad_ref[...] = a_d
    shift = jnp.max(a_s) + jnp.max(a_d)
    sh_ref[...] = jnp.full((1, 16), shift, jnp.float32)


def _project(x_pad, W, att_src, att_dst):
    return pl.pallas_call(
        _proj_body,
        out_shape=[
            jax.ShapeDtypeStruct((NP, H), jnp.float32),
            jax.ShapeDtypeStruct((NP, 1), jnp.float32),
            jax.ShapeDtypeStruct((NP, 1), jnp.float32),
            jax.ShapeDtypeStruct((1, 16), jnp.float32),
        ],
    )(x_pad, W, att_src.reshape(1, H), att_dst.reshape(1, H))


def _edge_body(h_hbm, as_hbm, ad_hbm, sh_hbm, idx_hbm,
               out_hbm, den_hbm,
               shift_v, idxb, av, dv, wbuf, rows_g,
               out_sp, den_sp, isem0, isem1, gsem0, gsem1):
    cid = lax.axis_index("c")
    sid = lax.axis_index("s")
    worker = cid * NS + sid
    isems = (isem0, isem1)
    gsems = (gsem0, gsem1)

    pltpu.sync_copy(sh_hbm, shift_v)
    shift_vec = shift_v[...]

    zf = jnp.zeros((16,), jnp.float32)

    @pl.loop(0, C)
    def _(r):
        for t in range(H // 16):
            rows_g[0, r, pl.ds(16 * t, 16)] = zf

    for t in range(C // 16):
        wbuf[pl.ds(16 * t, 16)] = zf

    row0 = sid * ROWS_PER_TILE
    off = 0
    for sz in _ZCHUNKS:
        pltpu.sync_copy(rows_g.at[0, pl.ds(0, sz)],
                        out_sp.at[pl.ds(row0 + off, sz)])
        pltpu.sync_copy(wbuf.at[pl.ds(0, sz)],
                        den_sp.at[pl.ds(row0 + off, sz)])
        off += sz
    plsc.subcore_barrier()

    pltpu.async_copy(idx_hbm.at[worker, pl.ds(0, 2)], idxb.at[0], isem0)
    pltpu.make_async_copy(idx_hbm.at[worker, pl.ds(0, 2)], idxb.at[0],
                          isem0).wait()
    pltpu.async_copy(h_hbm.at[idxb.at[0, 0]], rows_g.at[0], gsem0)
    pltpu.async_copy(idx_hbm.at[worker, pl.ds(2, 2)], idxb.at[1], isem1)

    @pl.loop(0, K, step=2)
    def _(k0):
        for b in range(2):
            o = 1 - b
            k = k0 + b
            @pl.when(k + 1 < K)
            def _():
                pltpu.make_async_copy(idx_hbm.at[worker, pl.ds(2 * k, 2)],
                                      idxb.at[o], isems[o]).wait()
                pltpu.async_copy(h_hbm.at[idxb.at[o, 0]], rows_g.at[o],
                                 gsems[o])

            pltpu.sync_copy(as_hbm.at[idxb.at[b, 0]], av)
            pltpu.sync_copy(ad_hbm.at[idxb.at[b, 1]], dv)
            for j in range(C // 16):
                sl = pl.ds(16 * j, 16)
                e = av[sl] + dv[sl]
                e = jnp.where(e < 0, e * jnp.float32(0.2), e)
                wbuf[sl] = jnp.exp(e - shift_vec)

            pltpu.make_async_copy(h_hbm.at[idxb.at[b, 0]], rows_g.at[b],
                                  gsems[b]).wait()

            @pl.loop(0, C)
            def _(r):
                wv = plsc.load_gather(wbuf, [jnp.full((16,), r, jnp.int32)])
                for t in range(H // 16):
                    sl = pl.ds(16 * t, 16)
                    rows_g[b, r, sl] = rows_g[b, r, sl] * wv

            pltpu.sync_copy(rows_g.at[b], out_sp.at[idxb.at[b, 1]], add=True)
            pltpu.sync_copy(wbuf, den_sp.at[idxb.at[b, 1]], add=True)

            @pl.when(k + 2 < K)
            def _():
                pltpu.async_copy(idx_hbm.at[worker, pl.ds(2 * (k + 2), 2)],
                                 idxb.at[b], isems[b])

    plsc.subcore_barrier()
    off = 0
    for sz in _ZCHUNKS:
        pltpu.sync_copy(out_sp.at[pl.ds(row0 + off, sz)],
                        out_hbm.at[cid, pl.ds(row0 + off, sz)])
        pltpu.sync_copy(den_sp.at[pl.ds(row0 + off, sz)],
                        den_hbm.at[cid, pl.ds(row0 + off, sz)])
        off += sz


def _edge_pass(h_pad, a_src, a_dst, shift, idx3):
    mesh = plsc.VectorSubcoreMesh(core_axis_name="c", subcore_axis_name="s")
    kern = pl.kernel(
        _edge_body,
        out_type=[
            jax.ShapeDtypeStruct((NC, NP, H), jnp.float32),
            jax.ShapeDtypeStruct((NC, NP), jnp.float32),
        ],
        mesh=mesh,
        compiler_params=pltpu.CompilerParams(needs_layout_passes=False),
        scratch_types=[
            pltpu.VMEM((16,), jnp.float32),
            pltpu.VMEM((2, 2, C), jnp.int32),
            pltpu.VMEM((C,), jnp.float32),
            pltpu.VMEM((C,), jnp.float32),
            pltpu.VMEM((C,), jnp.float32),
            pltpu.VMEM((2, C, H), jnp.float32),
            pltpu.VMEM_SHARED((NP, H), jnp.float32),
            pltpu.VMEM_SHARED((NP,), jnp.float32),
            pltpu.SemaphoreType.DMA,
            pltpu.SemaphoreType.DMA,
            pltpu.SemaphoreType.DMA,
            pltpu.SemaphoreType.DMA,
        ],
    )
    return kern(h_pad, a_src, a_dst, shift, idx3)


def _final_body(p_ref, d_ref, b_ref, g_ref, be_ref, m_ref, v_ref, o_ref):
    s = p_ref[0] + p_ref[1]
    den = d_ref[0] + d_ref[1] + 1e-16
    out = s / den + b_ref[...]
    out = jnp.maximum(out, 0.0)
    scale = g_ref[...] * jax.lax.rsqrt(v_ref[...] + 1e-5)
    o_ref[...] = (out - m_ref[...]) * scale + be_ref[...]


def _finalize(partials, denoms, bias, bn_gamma, bn_beta, bn_mean, bn_var):
    blk = 1000
    vec = lambda a: a.reshape(1, H)
    return pl.pallas_call(
        _final_body,
        grid=(N // blk,),
        in_specs=[
            pl.BlockSpec((NC, blk, H), lambda i: (0, i, 0)),
            pl.BlockSpec((NC, blk, 1), lambda i: (0, i, 0)),
            pl.BlockSpec((1, H), lambda i: (0, 0)),
            pl.BlockSpec((1, H), lambda i: (0, 0)),
            pl.BlockSpec((1, H), lambda i: (0, 0)),
            pl.BlockSpec((1, H), lambda i: (0, 0)),
            pl.BlockSpec((1, H), lambda i: (0, 0)),
        ],
        out_specs=pl.BlockSpec((blk, H), lambda i: (i, 0)),
        out_shape=jax.ShapeDtypeStruct((N, H), jnp.float32),
    )(partials, denoms.reshape(NC, NP, 1), vec(bias), vec(bn_gamma),
      vec(bn_beta), vec(bn_mean), vec(bn_var))


def kernel(x, edge_index, W, att_src, att_dst, bias, bn_gamma, bn_beta,
           bn_mean, bn_var):
    x_pad = jnp.pad(x, ((0, NP - N), (0, 0)))
    loop = jnp.arange(N, dtype=jnp.int32)
    pad = N + (jnp.arange(E_PAD - E_SELF, dtype=jnp.int32) % (NP - N))
    src_all = jnp.concatenate([edge_index[0], loop, pad])
    dst_all = jnp.concatenate([edge_index[1], loop, pad])
    idx3 = jnp.stack([src_all.reshape(NW, K, C), dst_all.reshape(NW, K, C)],
                     axis=2).reshape(NW, 2 * K, C)

    h_pad, a_src, a_dst, shift = _project(x_pad, W, att_src, att_dst)
    partials, denoms = _edge_pass(
        h_pad, a_src.reshape(NP), a_dst.reshape(NP), shift.reshape(16), idx3)
    return _finalize(partials, denoms, bias, bn_gamma, bn_beta, bn_mean,
                     bn_var)


# speedup vs baseline: 1.1161x; 1.1161x over previous
import functools

import jax
import jax.numpy as jnp
from jax import lax
from jax.experimental import pallas as pl
from jax.experimental.pallas import tpu as pltpu
from jax.experimental.pallas import tpu_sc as plsc

N = 10000
NP = 10240
H = 128
E_RAW = 320000
E_SELF = E_RAW + N
NC = 2
NS = 16
NW = NC * NS
C = 128
K = 82
E_PAD = NW * C * K
ROWS_PER_TILE = NP // NS
_ZCHUNKS = (128,) * (ROWS_PER_TILE // 128)


def _proj_body(x_ref, w_ref, asv_ref, adv_ref, h_ref, as_ref, ad_ref, sh_ref):
    h = jnp.dot(x_ref[...], w_ref[...], preferred_element_type=jnp.float32)
    h_ref[...] = h
    a_s = jnp.sum(h * asv_ref[...], axis=1, keepdims=True)
    a_d = jnp.sum(h * adv_ref[...], axis=1, keepdims=True)
    valid = jax.lax.broadcasted_iota(jnp.int32, (NP, 1), 0) < N
    a_s = jnp.where(valid, a_s, -1e30)
    a_d = jnp.where(valid, a_d, -1e30)
    as_ref[...] = a_s
    ad_ref[...] = a_d
    shift = jnp.max(a_s) + jnp.max(a_d)
    sh_ref[...] = jnp.full((1, 16), shift, jnp.float32)


def _project(x_pad, W, att_src, att_dst):
    return pl.pallas_call(
        _proj_body,
        out_shape=[
            jax.ShapeDtypeStruct((NP, H), jnp.float32),
            jax.ShapeDtypeStruct((NP, 1), jnp.float32),
            jax.ShapeDtypeStruct((NP, 1), jnp.float32),
            jax.ShapeDtypeStruct((1, 16), jnp.float32),
        ],
    )(x_pad, W, att_src.reshape(1, H), att_dst.reshape(1, H))


def _edge_body(h_hbm, as_hbm, ad_hbm, sh_hbm, idx_hbm,
               out_hbm, den_hbm,
               shift_v, idxb, av, dv, wbuf, rows_g,
               out_sp, den_sp, isem0, isem1, gsem0, gsem1):
    cid = lax.axis_index("c")
    sid = lax.axis_index("s")
    worker = cid * NS + sid
    isems = (isem0, isem1)
    gsems = (gsem0, gsem1)

    pltpu.sync_copy(sh_hbm, shift_v)
    shift_vec = shift_v[...]

    zf = jnp.zeros((16,), jnp.float32)

    @pl.loop(0, C)
    def _(r):
        for t in range(H // 16):
            rows_g[0, r, pl.ds(16 * t, 16)] = zf

    for t in range(C // 16):
        wbuf[pl.ds(16 * t, 16)] = zf

    row0 = sid * ROWS_PER_TILE
    off = 0
    for sz in _ZCHUNKS:
        pltpu.sync_copy(rows_g.at[0, pl.ds(0, sz)],
                        out_sp.at[pl.ds(row0 + off, sz)])
        pltpu.sync_copy(wbuf.at[pl.ds(0, sz)],
                        den_sp.at[pl.ds(row0 + off, sz)])
        off += sz
    plsc.subcore_barrier()

    pltpu.async_copy(idx_hbm.at[worker, pl.ds(0, 2)], idxb.at[0], isem0)
    pltpu.make_async_copy(idx_hbm.at[worker, pl.ds(0, 2)], idxb.at[0],
                          isem0).wait()
    pltpu.async_copy(h_hbm.at[idxb.at[0, 0]], rows_g.at[0], gsem0)
    pltpu.async_copy(idx_hbm.at[worker, pl.ds(2, 2)], idxb.at[1], isem1)

    @pl.loop(0, K, step=2)
    def _(k0):
        for b in range(2):
            o = 1 - b
            k = k0 + b
            @pl.when(k + 1 < K)
            def _():
                pltpu.make_async_copy(idx_hbm.at[worker, pl.ds(2 * k, 2)],
                                      idxb.at[o], isems[o]).wait()
                pltpu.async_copy(h_hbm.at[idxb.at[o, 0]], rows_g.at[o],
                                 gsems[o])

            pltpu.sync_copy(as_hbm.at[idxb.at[b, 0]], av)
            pltpu.sync_copy(ad_hbm.at[idxb.at[b, 1]], dv)
            for j in range(C // 16):
                sl = pl.ds(16 * j, 16)
                e = av[sl] + dv[sl]
                e = jnp.where(e < 0, e * jnp.float32(0.2), e)
                wbuf[sl] = jnp.exp(e - shift_vec)

            pltpu.make_async_copy(h_hbm.at[idxb.at[b, 0]], rows_g.at[b],
                                  gsems[b]).wait()

            @plsc.parallel_loop(0, C, unroll=8)
            def _(r):
                wv = plsc.load_gather(wbuf, [jnp.full((16,), r, jnp.int32)])
                for t in range(H // 16):
                    sl = pl.ds(16 * t, 16)
                    rows_g[b, r, sl] = rows_g[b, r, sl] * wv

            pltpu.sync_copy(rows_g.at[b], out_sp.at[idxb.at[b, 1]], add=True)
            pltpu.sync_copy(wbuf, den_sp.at[idxb.at[b, 1]], add=True)

            @pl.when(k + 2 < K)
            def _():
                pltpu.async_copy(idx_hbm.at[worker, pl.ds(2 * (k + 2), 2)],
                                 idxb.at[b], isems[b])

    plsc.subcore_barrier()
    off = 0
    for sz in _ZCHUNKS:
        pltpu.sync_copy(out_sp.at[pl.ds(row0 + off, sz)],
                        out_hbm.at[cid, pl.ds(row0 + off, sz)])
        pltpu.sync_copy(den_sp.at[pl.ds(row0 + off, sz)],
                        den_hbm.at[cid, pl.ds(row0 + off, sz)])
        off += sz


def _edge_pass(h_pad, a_src, a_dst, shift, idx3):
    mesh = plsc.VectorSubcoreMesh(core_axis_name="c", subcore_axis_name="s")
    kern = pl.kernel(
        _edge_body,
        out_type=[
            jax.ShapeDtypeStruct((NC, NP, H), jnp.float32),
            jax.ShapeDtypeStruct((NC, NP), jnp.float32),
        ],
        mesh=mesh,
        compiler_params=pltpu.CompilerParams(needs_layout_passes=False),
        scratch_types=[
            pltpu.VMEM((16,), jnp.float32),
            pltpu.VMEM((2, 2, C), jnp.int32),
            pltpu.VMEM((C,), jnp.float32),
            pltpu.VMEM((C,), jnp.float32),
            pltpu.VMEM((C,), jnp.float32),
            pltpu.VMEM((2, C, H), jnp.float32),
            pltpu.VMEM_SHARED((NP, H), jnp.float32),
            pltpu.VMEM_SHARED((NP,), jnp.float32),
            pltpu.SemaphoreType.DMA,
            pltpu.SemaphoreType.DMA,
            pltpu.SemaphoreType.DMA,
            pltpu.SemaphoreType.DMA,
        ],
    )
    return kern(h_pad, a_src, a_dst, shift, idx3)


def _final_body(p_ref, d_ref, b_ref, g_ref, be_ref, m_ref, v_ref, o_ref):
    s = p_ref[0] + p_ref[1]
    den = d_ref[0] + d_ref[1] + 1e-16
    out = s / den + b_ref[...]
    out = jnp.maximum(out, 0.0)
    scale = g_ref[...] * jax.lax.rsqrt(v_ref[...] + 1e-5)
    o_ref[...] = (out - m_ref[...]) * scale + be_ref[...]


def _finalize(partials, denoms, bias, bn_gamma, bn_beta, bn_mean, bn_var):
    blk = 1000
    vec = lambda a: a.reshape(1, H)
    return pl.pallas_call(
        _final_body,
        grid=(N // blk,),
        in_specs=[
            pl.BlockSpec((NC, blk, H), lambda i: (0, i, 0)),
            pl.BlockSpec((NC, blk, 1), lambda i: (0, i, 0)),
            pl.BlockSpec((1, H), lambda i: (0, 0)),
            pl.BlockSpec((1, H), lambda i: (0, 0)),
            pl.BlockSpec((1, H), lambda i: (0, 0)),
            pl.BlockSpec((1, H), lambda i: (0, 0)),
            pl.BlockSpec((1, H), lambda i: (0, 0)),
        ],
        out_specs=pl.BlockSpec((blk, H), lambda i: (i, 0)),
        out_shape=jax.ShapeDtypeStruct((N, H), jnp.float32),
    )(partials, denoms.reshape(NC, NP, 1), vec(bias), vec(bn_gamma),
      vec(bn_beta), vec(bn_mean), vec(bn_var))


def kernel(x, edge_index, W, att_src, att_dst, bias, bn_gamma, bn_beta,
           bn_mean, bn_var):
    x_pad = jnp.pad(x, ((0, NP - N), (0, 0)))
    loop = jnp.arange(N, dtype=jnp.int32)
    pad = N + (jnp.arange(E_PAD - E_SELF, dtype=jnp.int32) % (NP - N))
    src_all = jnp.concatenate([edge_index[0], loop, pad])
    dst_all = jnp.concatenate([edge_index[1], loop, pad])
    idx3 = jnp.stack([src_all.reshape(NW, K, C), dst_all.reshape(NW, K, C)],
                     axis=2).reshape(NW, 2 * K, C)

    h_pad, a_src, a_dst, shift = _project(x_pad, W, att_src, att_dst)
    partials, denoms = _edge_pass(
        h_pad, a_src.reshape(NP), a_dst.reshape(NP), shift.reshape(16), idx3)
    return _finalize(partials, denoms, bias, bn_gamma, bn_beta, bn_mean,
                     bn_var)


# speedup vs baseline: 1.3316x; 1.1931x over previous
import functools

import jax
import jax.numpy as jnp
from jax import lax
from jax.experimental import pallas as pl
from jax.experimental.pallas import tpu as pltpu
from jax.experimental.pallas import tpu_sc as plsc

N = 10000
NP = 10240
H = 128
E_RAW = 320000
E_SELF = E_RAW + N
NC = 2
NS = 16
NW = NC * NS
C = 112
K = 93
E_PAD = NW * C * K
ROWS_PER_TILE = NP // NS
_ZCHUNKS = [(i * C, min(C, ROWS_PER_TILE - i * C))
            for i in range(-(-ROWS_PER_TILE // C))]
_OCHUNKS = [(i * 128, 128) for i in range(ROWS_PER_TILE // 128)]


def _proj_body(x_ref, w_ref, asv_ref, adv_ref, h_ref, as_ref, ad_ref, sh_ref):
    h = jnp.dot(x_ref[...], w_ref[...], preferred_element_type=jnp.float32)
    h_ref[...] = h
    a_s = jnp.sum(h * asv_ref[...], axis=1, keepdims=True)
    a_d = jnp.sum(h * adv_ref[...], axis=1, keepdims=True)
    valid = jax.lax.broadcasted_iota(jnp.int32, (NP, 1), 0) < N
    a_s = jnp.where(valid, a_s, -1e30)
    a_d = jnp.where(valid, a_d, -1e30)
    as_ref[...] = a_s
    ad_ref[...] = a_d
    shift = jnp.max(a_s) + jnp.max(a_d)
    sh_ref[...] = jnp.full((1, 16), shift, jnp.float32)


def _project(x_pad, W, att_src, att_dst):
    return pl.pallas_call(
        _proj_body,
        out_shape=[
            jax.ShapeDtypeStruct((NP, H), jnp.float32),
            jax.ShapeDtypeStruct((NP, 1), jnp.float32),
            jax.ShapeDtypeStruct((NP, 1), jnp.float32),
            jax.ShapeDtypeStruct((1, 16), jnp.float32),
        ],
    )(x_pad, W, att_src.reshape(1, H), att_dst.reshape(1, H))


def _edge_body(h_hbm, as_hbm, ad_hbm, sh_hbm, idx_hbm,
               out_hbm, den_hbm,
               shift_v, idxb, av, dv, wbuf, rows_g, out_sp, den_sp,
               isem0, isem1, isem2, gsem0, gsem1, gsem2,
               ssem0, ssem1, ssem2):
    cid = lax.axis_index("c")
    sid = lax.axis_index("s")
    worker = cid * NS + sid
    isems = (isem0, isem1, isem2)
    gsems = (gsem0, gsem1, gsem2)
    ssems = (ssem0, ssem1, ssem2)

    def idx_copy(k, s, sem):
        return pltpu.make_async_copy(idx_hbm.at[worker, pl.ds(2 * k, 2)],
                                     idxb.at[s], sem)

    def row_gather(s, sem):
        return pltpu.make_async_copy(h_hbm.at[idxb.at[s, 0]], rows_g.at[s],
                                     sem)

    def row_scatter(s, sem):
        return pltpu.make_async_copy(rows_g.at[s], out_sp.at[idxb.at[s, 1]],
                                     sem)

    pltpu.sync_copy(sh_hbm, shift_v)
    shift_vec = shift_v[...]

    zf = jnp.zeros((16,), jnp.float32)

    @pl.loop(0, C)
    def _(r):
        for t in range(H // 16):
            rows_g[0, r, pl.ds(16 * t, 16)] = zf

    for t in range(C // 16):
        wbuf[pl.ds(16 * t, 16)] = zf

    row0 = sid * ROWS_PER_TILE
    for off, sz in _ZCHUNKS:
        pltpu.sync_copy(rows_g.at[0, pl.ds(0, sz)],
                        out_sp.at[pl.ds(row0 + off, sz)])
        pltpu.sync_copy(wbuf.at[pl.ds(0, sz)],
                        den_sp.at[pl.ds(row0 + off, sz)])
    plsc.subcore_barrier()

    idx_copy(0, 0, isem0).start()
    idx_copy(1, 1, isem1).start()
    idx_copy(0, 0, isem0).wait()
    row_gather(0, gsem0).start()

    @pl.loop(0, K, step=3)
    def _(k0):
        for b in range(3):
            j = k0 + b
            s = b
            s1 = (b + 1) % 3
            sm1 = (b + 2) % 3
            pltpu.sync_copy(as_hbm.at[idxb.at[s, 0]], av)
            pltpu.sync_copy(ad_hbm.at[idxb.at[s, 1]], dv)
            for t in range(C // 16):
                sl = pl.ds(16 * t, 16)
                e = av[sl] + dv[sl]
                e = jnp.where(e < 0, e * jnp.float32(0.2), e)
                wbuf[sl] = jnp.exp(e - shift_vec)

            row_gather(s, gsems[s]).wait()

            @plsc.parallel_loop(0, C, unroll=8)
            def _(r):
                wv = plsc.load_gather(wbuf, [jnp.full((16,), r, jnp.int32)])
                for t in range(H // 16):
                    sl = pl.ds(16 * t, 16)
                    rows_g[s, r, sl] = rows_g[s, r, sl] * wv

            @pl.when(j + 1 < K)
            def _():
                idx_copy(j + 1, s1, isems[s1]).wait()
                row_gather(s1, gsems[s1]).start()

            row_scatter(s, ssems[s]).start(add=True)
            pltpu.sync_copy(wbuf, den_sp.at[idxb.at[s, 1]], add=True)

            @pl.when(j >= 1)
            def _():
                @pl.when(j + 2 < K)
                def _():
                    row_scatter(sm1, ssems[sm1]).wait()
                    idx_copy(j + 2, sm1, isems[sm1]).start()

            @pl.when(j == 0)
            def _():
                idx_copy(2, 2, isem2).start()

    row_scatter(0, ssem0).wait()
    row_scatter(1, ssem1).wait()
    row_scatter(2, ssem2).wait()
    plsc.subcore_barrier()
    for off, sz in _OCHUNKS:
        pltpu.sync_copy(out_sp.at[pl.ds(row0 + off, sz)],
                        out_hbm.at[cid, pl.ds(row0 + off, sz)])
        pltpu.sync_copy(den_sp.at[pl.ds(row0 + off, sz)],
                        den_hbm.at[pl.ds(cid * NP + row0 + off, sz)])


def _edge_pass(h_pad, a_src, a_dst, shift, idx3):
    mesh = plsc.VectorSubcoreMesh(core_axis_name="c", subcore_axis_name="s")
    kern = pl.kernel(
        _edge_body,
        out_type=[
            jax.ShapeDtypeStruct((NC, NP, H), jnp.float32),
            jax.ShapeDtypeStruct((NC * NP,), jnp.float32),
        ],
        mesh=mesh,
        compiler_params=pltpu.CompilerParams(needs_layout_passes=False),
        scratch_types=[
            pltpu.VMEM((16,), jnp.float32),
            pltpu.VMEM((3, 2, C), jnp.int32),
            pltpu.VMEM((C,), jnp.float32),
            pltpu.VMEM((C,), jnp.float32),
            pltpu.VMEM((C,), jnp.float32),
            pltpu.VMEM((3, C, H), jnp.float32),
            pltpu.VMEM_SHARED((NP, H), jnp.float32),
            pltpu.VMEM_SHARED((NP,), jnp.float32),
        ] + [pltpu.SemaphoreType.DMA] * 9,
    )
    return kern(h_pad, a_src, a_dst, shift, idx3)


def _final_body(p_ref, d_ref, b_ref, g_ref, be_ref, m_ref, v_ref, o_ref):
    s = p_ref[0] + p_ref[1]
    den = d_ref[0] + d_ref[1] + 1e-16
    out = s / den + b_ref[...]
    out = jnp.maximum(out, 0.0)
    scale = g_ref[...] * jax.lax.rsqrt(v_ref[...] + 1e-5)
    o_ref[...] = (out - m_ref[...]) * scale + be_ref[...]


def _finalize(partials, denoms, bias, bn_gamma, bn_beta, bn_mean, bn_var):
    blk = 1000
    vec = lambda a: a.reshape(1, H)
    return pl.pallas_call(
        _final_body,
        grid=(N // blk,),
        in_specs=[
            pl.BlockSpec((NC, blk, H), lambda i: (0, i, 0)),
            pl.BlockSpec((NC, blk, 1), lambda i: (0, i, 0)),
            pl.BlockSpec((1, H), lambda i: (0, 0)),
            pl.BlockSpec((1, H), lambda i: (0, 0)),
            pl.BlockSpec((1, H), lambda i: (0, 0)),
            pl.BlockSpec((1, H), lambda i: (0, 0)),
            pl.BlockSpec((1, H), lambda i: (0, 0)),
        ],
        out_specs=pl.BlockSpec((blk, H), lambda i: (i, 0)),
        out_shape=jax.ShapeDtypeStruct((N, H), jnp.float32),
    )(partials, denoms.reshape(NC, NP, 1), vec(bias), vec(bn_gamma),
      vec(bn_beta), vec(bn_mean), vec(bn_var))


def kernel(x, edge_index, W, att_src, att_dst, bias, bn_gamma, bn_beta,
           bn_mean, bn_var):
    x_pad = jnp.pad(x, ((0, NP - N), (0, 0)))
    loop = jnp.arange(N, dtype=jnp.int32)
    pad = N + (jnp.arange(E_PAD - E_SELF, dtype=jnp.int32) % (NP - N))
    src_all = jnp.concatenate([edge_index[0], loop, pad])
    dst_all = jnp.concatenate([edge_index[1], loop, pad])
    idx3 = jnp.stack([src_all.reshape(NW, K, C), dst_all.reshape(NW, K, C)],
                     axis=2).reshape(NW, 2 * K, C)

    h_pad, a_src, a_dst, shift = _project(x_pad, W, att_src, att_dst)
    partials, denoms = _edge_pass(
        h_pad, a_src.reshape(NP), a_dst.reshape(NP), shift.reshape(16), idx3)
    return _finalize(partials, denoms, bias, bn_gamma, bn_beta, bn_mean,
                     bn_var)


# speedup vs baseline: 1.5453x; 1.1605x over previous
import functools

import jax
import jax.numpy as jnp
from jax import lax
from jax.experimental import pallas as pl
from jax.experimental.pallas import tpu as pltpu
from jax.experimental.pallas import tpu_sc as plsc

N = 10000
NP = 10240
H = 128
E_RAW = 320000
E_SELF = E_RAW + N
NC = 2
NS = 16
NW = NC * NS
C = 112
K = 93
E_PAD = NW * C * K
ROWS_PER_TILE = NP // NS
_ZCHUNKS = [(i * C, min(C, ROWS_PER_TILE - i * C))
            for i in range(-(-ROWS_PER_TILE // C))]
_OCHUNKS = [(i * 128, 128) for i in range(ROWS_PER_TILE // 128)]


def _proj_body(x_ref, w_ref, asv_ref, adv_ref, h_ref, as_ref, ad_ref, sh_ref):
    h = jnp.dot(x_ref[...], w_ref[...], preferred_element_type=jnp.float32)
    h_ref[...] = h
    a_s = jnp.sum(h * asv_ref[...], axis=1, keepdims=True)
    a_d = jnp.sum(h * adv_ref[...], axis=1, keepdims=True)
    valid = jax.lax.broadcasted_iota(jnp.int32, (NP, 1), 0) < N
    a_s = jnp.where(valid, a_s, -1e30)
    a_d = jnp.where(valid, a_d, -1e30)
    as_ref[...] = a_s
    ad_ref[...] = a_d
    shift = jnp.max(a_s) + jnp.max(a_d)
    sh_ref[...] = jnp.full((1, 16), shift, jnp.float32)


def _project(x_pad, W, att_src, att_dst):
    return pl.pallas_call(
        _proj_body,
        out_shape=[
            jax.ShapeDtypeStruct((NP, H), jnp.float32),
            jax.ShapeDtypeStruct((NP, 1), jnp.float32),
            jax.ShapeDtypeStruct((NP, 1), jnp.float32),
            jax.ShapeDtypeStruct((1, 16), jnp.float32),
        ],
    )(x_pad, W, att_src.reshape(1, H), att_dst.reshape(1, H))


def _edge_body(h_hbm, asd_hbm, sh_hbm, idx_hbm,
               out_hbm, den_hbm,
               shift_v, idxb, avdv, wbuf, rows_g, out_sp, den_sp,
               isem0, isem1, isem2, gsem0, gsem1, gsem2,
               ssem0, ssem1, ssem2, asem0, asem1, asem2,
               wsem0, wsem1, wsem2):
    cid = lax.axis_index("c")
    sid = lax.axis_index("s")
    worker = cid * NS + sid
    isems = (isem0, isem1, isem2)
    gsems = (gsem0, gsem1, gsem2)
    ssems = (ssem0, ssem1, ssem2)
    asems = (asem0, asem1, asem2)
    wsems = (wsem0, wsem1, wsem2)

    def idx_copy(k, s, sem):
        return pltpu.make_async_copy(idx_hbm.at[worker, k], idxb.at[s], sem)

    def row_gather(s, sem):
        return pltpu.make_async_copy(h_hbm.at[idxb.at[s, 0]], rows_g.at[s],
                                     sem)

    def logit_gather_a(s, sem):
        return pltpu.make_async_copy(asd_hbm.at[idxb.at[s, 0]],
                                     avdv.at[s, 0], sem)

    def logit_gather_b(s, sem):
        return pltpu.make_async_copy(asd_hbm.at[idxb.at[s, 1]],
                                     avdv.at[s, 1], sem)

    def row_scatter(s, sem):
        return pltpu.make_async_copy(rows_g.at[s], out_sp.at[idxb.at[s, 2]],
                                     sem)

    def den_scatter(s, sem):
        return pltpu.make_async_copy(wbuf.at[s], den_sp.at[idxb.at[s, 2]],
                                     sem)

    pltpu.sync_copy(sh_hbm, shift_v)
    shift_vec = shift_v[...]

    zf = jnp.zeros((16,), jnp.float32)

    @pl.loop(0, C)
    def _(r):
        for t in range(H // 16):
            rows_g[0, r, pl.ds(16 * t, 16)] = zf

    for t in range(C // 16):
        wbuf[0, pl.ds(16 * t, 16)] = zf

    row0 = sid * ROWS_PER_TILE
    for off, sz in _ZCHUNKS:
        pltpu.sync_copy(rows_g.at[0, pl.ds(0, sz)],
                        out_sp.at[pl.ds(row0 + off, sz)])
        pltpu.sync_copy(wbuf.at[0, pl.ds(0, sz)],
                        den_sp.at[pl.ds(row0 + off, sz)])
    plsc.subcore_barrier()

    idx_copy(0, 0, isem0).start()
    idx_copy(1, 1, isem1).start()
    idx_copy(0, 0, isem0).wait()
    row_gather(0, gsem0).start()
    logit_gather_a(0, asem0).start()
    logit_gather_b(0, asem0).start()

    @pl.loop(0, K, step=3)
    def _(k0):
        for b in range(3):
            j = k0 + b
            s = b
            s1 = (b + 1) % 3
            sm1 = (b + 2) % 3
            logit_gather_a(s, asems[s]).wait()
            logit_gather_b(s, asems[s]).wait()
            for t in range(C // 16):
                sl = pl.ds(16 * t, 16)
                e = avdv[s, 0, sl] + avdv[s, 1, sl]
                e = jnp.where(e < 0, e * jnp.float32(0.2), e)
                wbuf[s, sl] = jnp.exp(e - shift_vec)

            row_gather(s, gsems[s]).wait()

            @plsc.parallel_loop(0, C, unroll=8)
            def _(r):
                wv = plsc.load_gather(wbuf.at[s],
                                      [jnp.full((16,), r, jnp.int32)])
                for t in range(H // 16):
                    sl = pl.ds(16 * t, 16)
                    rows_g[s, r, sl] = rows_g[s, r, sl] * wv

            @pl.when(j + 1 < K)
            def _():
                idx_copy(j + 1, s1, isems[s1]).wait()
                row_gather(s1, gsems[s1]).start()
                logit_gather_a(s1, asems[s1]).start()
                logit_gather_b(s1, asems[s1]).start()

            row_scatter(s, ssems[s]).start(add=True)
            den_scatter(s, wsems[s]).start(add=True)

            @pl.when(j >= 1)
            def _():
                @pl.when(j + 2 < K)
                def _():
                    row_scatter(sm1, ssems[sm1]).wait()
                    den_scatter(sm1, wsems[sm1]).wait()
                    idx_copy(j + 2, sm1, isems[sm1]).start()

            @pl.when(j == 0)
            def _():
                idx_copy(2, 2, isem2).start()

    for s in range(3):
        row_scatter(s, ssems[s]).wait()
        den_scatter(s, wsems[s]).wait()
    plsc.subcore_barrier()
    for off, sz in _OCHUNKS:
        pltpu.sync_copy(out_sp.at[pl.ds(row0 + off, sz)],
                        out_hbm.at[cid, pl.ds(row0 + off, sz)])
        pltpu.sync_copy(den_sp.at[pl.ds(row0 + off, sz)],
                        den_hbm.at[pl.ds(cid * NP + row0 + off, sz)])


def _edge_pass(h_pad, asd, shift, idx3):
    mesh = plsc.VectorSubcoreMesh(core_axis_name="c", subcore_axis_name="s")
    kern = pl.kernel(
        _edge_body,
        out_type=[
            jax.ShapeDtypeStruct((NC, NP, H), jnp.float32),
            jax.ShapeDtypeStruct((NC * NP,), jnp.float32),
        ],
        mesh=mesh,
        compiler_params=pltpu.CompilerParams(needs_layout_passes=False),
        scratch_types=[
            pltpu.VMEM((16,), jnp.float32),
            pltpu.VMEM((3, 3, C), jnp.int32),
            pltpu.VMEM((3, 2, C), jnp.float32),
            pltpu.VMEM((3, C), jnp.float32),
            pltpu.VMEM((3, C, H), jnp.float32),
            pltpu.VMEM_SHARED((NP, H), jnp.float32),
            pltpu.VMEM_SHARED((NP,), jnp.float32),
        ] + [pltpu.SemaphoreType.DMA] * 15,
    )
    return kern(h_pad, asd, shift, idx3)


def _final_body(p_ref, d_ref, b_ref, g_ref, be_ref, m_ref, v_ref, o_ref):
    s = p_ref[0] + p_ref[1]
    den = d_ref[0] + d_ref[1] + 1e-16
    out = s / den + b_ref[...]
    out = jnp.maximum(out, 0.0)
    scale = g_ref[...] * jax.lax.rsqrt(v_ref[...] + 1e-5)
    o_ref[...] = (out - m_ref[...]) * scale + be_ref[...]


def _finalize(partials, denoms, bias, bn_gamma, bn_beta, bn_mean, bn_var):
    blk = 1000
    vec = lambda a: a.reshape(1, H)
    return pl.pallas_call(
        _final_body,
        grid=(N // blk,),
        in_specs=[
            pl.BlockSpec((NC, blk, H), lambda i: (0, i, 0)),
            pl.BlockSpec((NC, blk, 1), lambda i: (0, i, 0)),
            pl.BlockSpec((1, H), lambda i: (0, 0)),
            pl.BlockSpec((1, H), lambda i: (0, 0)),
            pl.BlockSpec((1, H), lambda i: (0, 0)),
            pl.BlockSpec((1, H), lambda i: (0, 0)),
            pl.BlockSpec((1, H), lambda i: (0, 0)),
        ],
        out_specs=pl.BlockSpec((blk, H), lambda i: (i, 0)),
        out_shape=jax.ShapeDtypeStruct((N, H), jnp.float32),
    )(partials, denoms.reshape(NC, NP, 1), vec(bias), vec(bn_gamma),
      vec(bn_beta), vec(bn_mean), vec(bn_var))


def kernel(x, edge_index, W, att_src, att_dst, bias, bn_gamma, bn_beta,
           bn_mean, bn_var):
    x_pad = jnp.pad(x, ((0, NP - N), (0, 0)))
    loop = jnp.arange(N, dtype=jnp.int32)
    pad = N + (jnp.arange(E_PAD - E_SELF, dtype=jnp.int32) % (NP - N))
    src_all = jnp.concatenate([edge_index[0], loop, pad])
    dst_all = jnp.concatenate([edge_index[1], loop, pad])
    idx3 = jnp.stack([src_all.reshape(NW, K, C),
                      dst_all.reshape(NW, K, C) + NP,
                      dst_all.reshape(NW, K, C)], axis=2)

    h_pad, a_src, a_dst, shift = _project(x_pad, W, att_src, att_dst)
    asd = jnp.concatenate([a_src.reshape(NP), a_dst.reshape(NP)])
    partials, denoms = _edge_pass(h_pad, asd, shift.reshape(16), idx3)
    return _finalize(partials, denoms, bias, bn_gamma, bn_beta, bn_mean,
                     bn_var)


# speedup vs baseline: 1.8419x; 1.1919x over previous
import functools

import jax
import jax.numpy as jnp
from jax import lax
from jax.experimental import pallas as pl
from jax.experimental.pallas import tpu as pltpu
from jax.experimental.pallas import tpu_sc as plsc

N = 10000
NP = 10240
H = 128
E_RAW = 320000
E_SELF = E_RAW + N
NC = 2
NS = 16
NW = NC * NS
C = 112
K = 93
E_PAD = NW * C * K
ROWS_PER_TILE = NP // NS
_ZCHUNKS = [(i * C, min(C, ROWS_PER_TILE - i * C))
            for i in range(-(-ROWS_PER_TILE // C))]
_OCHUNKS = [(i * 128, 128) for i in range(ROWS_PER_TILE // 128)]


def _proj_body(x_ref, w_ref, asv_ref, adv_ref, h_ref, as_ref, ad_ref, sh_ref):
    h = jnp.dot(x_ref[...], w_ref[...], preferred_element_type=jnp.float32)
    h_ref[...] = h
    a_s = jnp.sum(h * asv_ref[...], axis=1, keepdims=True)
    a_d = jnp.sum(h * adv_ref[...], axis=1, keepdims=True)
    valid = jax.lax.broadcasted_iota(jnp.int32, (NP, 1), 0) < N
    a_s = jnp.where(valid, a_s, -1e30)
    a_d = jnp.where(valid, a_d, -1e30)
    as_ref[...] = a_s
    ad_ref[...] = a_d
    shift = jnp.max(a_s) + jnp.max(a_d)
    sh_ref[...] = jnp.full((1, 16), shift, jnp.float32)


def _project(x_pad, W, att_src, att_dst):
    return pl.pallas_call(
        _proj_body,
        out_shape=[
            jax.ShapeDtypeStruct((NP, H), jnp.float32),
            jax.ShapeDtypeStruct((NP, 1), jnp.float32),
            jax.ShapeDtypeStruct((NP, 1), jnp.float32),
            jax.ShapeDtypeStruct((1, 16), jnp.float32),
        ],
    )(x_pad, W, att_src.reshape(1, H), att_dst.reshape(1, H))


def _edge_body(h_hbm, asd_hbm, sh_hbm, idx_hbm,
               out_hbm, den_hbm,
               shift_v, idxb, avdv, wbuf, rows_g, out_sp, den_sp,
               isem0, isem1, isem2, gsem0, gsem1, gsem2,
               ssem0, ssem1, ssem2, asem0, asem1, asem2,
               wsem0, wsem1, wsem2):
    cid = lax.axis_index("c")
    sid = lax.axis_index("s")
    worker = cid * NS + sid
    isems = (isem0, isem1, isem2)
    gsems = (gsem0, gsem1, gsem2)
    ssems = (ssem0, ssem1, ssem2)
    asems = (asem0, asem1, asem2)
    wsems = (wsem0, wsem1, wsem2)

    def idx_copy(k, s, sem):
        return pltpu.make_async_copy(idx_hbm.at[worker, k], idxb.at[s], sem)

    def row_gather(s, sem):
        return pltpu.make_async_copy(h_hbm.at[idxb.at[s, 0]], rows_g.at[s],
                                     sem)

    def logit_gather_a(s, sem):
        return pltpu.make_async_copy(asd_hbm.at[idxb.at[s, 0]],
                                     avdv.at[s, 0], sem)

    def logit_gather_b(s, sem):
        return pltpu.make_async_copy(asd_hbm.at[idxb.at[s, 1]],
                                     avdv.at[s, 1], sem)

    def row_scatter(s, sem):
        return pltpu.make_async_copy(rows_g.at[s], out_sp.at[idxb.at[s, 2]],
                                     sem)

    def den_scatter(s, sem):
        return pltpu.make_async_copy(wbuf.at[s], den_sp.at[idxb.at[s, 2]],
                                     sem)

    pltpu.sync_copy(sh_hbm, shift_v)
    shift_vec = shift_v[...]

    zf = jnp.zeros((16,), jnp.float32)

    @pl.loop(0, C)
    def _(r):
        for t in range(H // 16):
            rows_g[0, r, pl.ds(16 * t, 16)] = zf

    for t in range(C // 16):
        wbuf[0, pl.ds(16 * t, 16)] = zf

    row0 = sid * ROWS_PER_TILE
    for off, sz in _ZCHUNKS:
        pltpu.sync_copy(rows_g.at[0, pl.ds(0, sz)],
                        out_sp.at[pl.ds(row0 + off, sz)])
        pltpu.sync_copy(wbuf.at[0, pl.ds(0, sz)],
                        den_sp.at[pl.ds(row0 + off, sz)])
    plsc.subcore_barrier()

    idx_copy(0, 0, isem0).start()
    idx_copy(1, 1, isem1).start()
    idx_copy(0, 0, isem0).wait()
    row_gather(0, gsem0).start()
    logit_gather_a(0, asem0).start()
    logit_gather_b(0, asem0).start()

    @pl.loop(0, K, step=3)
    def _(k0):
        for b in range(3):
            j = k0 + b
            s = b
            s1 = (b + 1) % 3
            sm1 = (b + 2) % 3
            logit_gather_a(s, asems[s]).wait()
            logit_gather_b(s, asems[s]).wait()
            for t in range(C // 16):
                sl = pl.ds(16 * t, 16)
                e = avdv[s, 0, sl] + avdv[s, 1, sl]
                e = jnp.where(e < 0, e * jnp.float32(0.2), e)
                wbuf[s, sl] = jnp.exp(e - shift_vec)

            @pl.when(j + 1 < K)
            def _():
                idx_copy(j + 1, s1, isems[s1]).wait()
                row_gather(s1, gsems[s1]).start()
                logit_gather_a(s1, asems[s1]).start()
                logit_gather_b(s1, asems[s1]).start()

            row_gather(s, gsems[s]).wait()

            @plsc.parallel_loop(0, C, unroll=8)
            def _(r):
                wv = plsc.load_gather(wbuf.at[s],
                                      [jnp.full((16,), r, jnp.int32)])
                for t in range(H // 16):
                    sl = pl.ds(16 * t, 16)
                    rows_g[s, r, sl] = rows_g[s, r, sl] * wv

            row_scatter(s, ssems[s]).start(add=True)
            den_scatter(s, wsems[s]).start(add=True)

            @pl.when(j >= 1)
            def _():
                @pl.when(j + 2 < K)
                def _():
                    row_scatter(sm1, ssems[sm1]).wait()
                    den_scatter(sm1, wsems[sm1]).wait()
                    idx_copy(j + 2, sm1, isems[sm1]).start()

            @pl.when(j == 0)
            def _():
                idx_copy(2, 2, isem2).start()

    for s in range(3):
        row_scatter(s, ssems[s]).wait()
        den_scatter(s, wsems[s]).wait()
    plsc.subcore_barrier()
    for off, sz in _OCHUNKS:
        pltpu.sync_copy(out_sp.at[pl.ds(row0 + off, sz)],
                        out_hbm.at[cid, pl.ds(row0 + off, sz)])
        pltpu.sync_copy(den_sp.at[pl.ds(row0 + off, sz)],
                        den_hbm.at[pl.ds(cid * NP + row0 + off, sz)])


def _edge_pass(h_pad, asd, shift, idx3):
    mesh = plsc.VectorSubcoreMesh(core_axis_name="c", subcore_axis_name="s")
    kern = pl.kernel(
        _edge_body,
        out_type=[
            jax.ShapeDtypeStruct((NC, NP, H), jnp.float32),
            jax.ShapeDtypeStruct((NC * NP,), jnp.float32),
        ],
        mesh=mesh,
        compiler_params=pltpu.CompilerParams(needs_layout_passes=False),
        scratch_types=[
            pltpu.VMEM((16,), jnp.float32),
            pltpu.VMEM((3, 3, C), jnp.int32),
            pltpu.VMEM((3, 2, C), jnp.float32),
            pltpu.VMEM((3, C), jnp.float32),
            pltpu.VMEM((3, C, H), jnp.float32),
            pltpu.VMEM_SHARED((NP, H), jnp.float32),
            pltpu.VMEM_SHARED((NP,), jnp.float32),
        ] + [pltpu.SemaphoreType.DMA] * 15,
    )
    return kern(h_pad, asd, shift, idx3)


def _final_body(p_ref, d_ref, b_ref, g_ref, be_ref, m_ref, v_ref, o_ref):
    s = p_ref[0] + p_ref[1]
    den = d_ref[0] + d_ref[1] + 1e-16
    out = s / den + b_ref[...]
    out = jnp.maximum(out, 0.0)
    scale = g_ref[...] * jax.lax.rsqrt(v_ref[...] + 1e-5)
    o_ref[...] = (out - m_ref[...]) * scale + be_ref[...]


def _finalize(partials, denoms, bias, bn_gamma, bn_beta, bn_mean, bn_var):
    blk = 1000
    vec = lambda a: a.reshape(1, H)
    return pl.pallas_call(
        _final_body,
        grid=(N // blk,),
        in_specs=[
            pl.BlockSpec((NC, blk, H), lambda i: (0, i, 0)),
            pl.BlockSpec((NC, blk, 1), lambda i: (0, i, 0)),
            pl.BlockSpec((1, H), lambda i: (0, 0)),
            pl.BlockSpec((1, H), lambda i: (0, 0)),
            pl.BlockSpec((1, H), lambda i: (0, 0)),
            pl.BlockSpec((1, H), lambda i: (0, 0)),
            pl.BlockSpec((1, H), lambda i: (0, 0)),
        ],
        out_specs=pl.BlockSpec((blk, H), lambda i: (i, 0)),
        out_shape=jax.ShapeDtypeStruct((N, H), jnp.float32),
    )(partials, denoms.reshape(NC, NP, 1), vec(bias), vec(bn_gamma),
      vec(bn_beta), vec(bn_mean), vec(bn_var))


def kernel(x, edge_index, W, att_src, att_dst, bias, bn_gamma, bn_beta,
           bn_mean, bn_var):
    x_pad = jnp.pad(x, ((0, NP - N), (0, 0)))
    loop = jnp.arange(N, dtype=jnp.int32)
    pad = N + (jnp.arange(E_PAD - E_SELF, dtype=jnp.int32) % (NP - N))
    src_all = jnp.concatenate([edge_index[0], loop, pad])
    dst_all = jnp.concatenate([edge_index[1], loop, pad])
    idx3 = jnp.stack([src_all.reshape(NW, K, C),
                      dst_all.reshape(NW, K, C) + NP,
                      dst_all.reshape(NW, K, C)], axis=2)

    h_pad, a_src, a_dst, shift = _project(x_pad, W, att_src, att_dst)
    asd = jnp.concatenate([a_src.reshape(NP), a_dst.reshape(NP)])
    partials, denoms = _edge_pass(h_pad, asd, shift.reshape(16), idx3)
    return _finalize(partials, denoms, bias, bn_gamma, bn_beta, bn_mean,
                     bn_var)


# speedup vs baseline: 2.1418x; 1.1628x over previous
import functools

import jax
import jax.numpy as jnp
from jax import lax
from jax.experimental import pallas as pl
from jax.experimental.pallas import tpu as pltpu
from jax.experimental.pallas import tpu_sc as plsc

N = 10000
NP = 10240
H = 128
E_RAW = 320000
NC = 2
NS = 16
NW = NC * NS
C = 112
K = 90
E_PAD = NW * C * K
ROWS_PER_TILE = NP // NS
_ZCHUNKS = [(i * C, min(C, ROWS_PER_TILE - i * C))
            for i in range(-(-ROWS_PER_TILE // C))]
_OCHUNKS = [(i * 128, 128) for i in range(ROWS_PER_TILE // 128)]


def _proj_body(x_ref, w_ref, asv_ref, adv_ref, h_ref, as_ref, ad_ref, sh_ref):
    h = jnp.dot(x_ref[...], w_ref[...], preferred_element_type=jnp.float32)
    h_ref[pl.ds(0, N), :] = h
    h_ref[pl.ds(N, NP - N), :] = jnp.zeros((NP - N, H), jnp.float32)
    a_s = jnp.sum(h * asv_ref[...], axis=1, keepdims=True)
    a_d = jnp.sum(h * adv_ref[...], axis=1, keepdims=True)
    as_ref[pl.ds(0, N), :] = a_s
    ad_ref[pl.ds(0, N), :] = a_d
    pad = jnp.full((NP - N, 1), -1e30, jnp.float32)
    as_ref[pl.ds(N, NP - N), :] = pad
    ad_ref[pl.ds(N, NP - N), :] = pad
    shift = jnp.max(a_s) + jnp.max(a_d)
    sh_ref[...] = jnp.full((1, 16), shift, jnp.float32)


def _project(x, W, att_src, att_dst):
    return pl.pallas_call(
        _proj_body,
        out_shape=[
            jax.ShapeDtypeStruct((NP, H), jnp.float32),
            jax.ShapeDtypeStruct((NP, 1), jnp.float32),
            jax.ShapeDtypeStruct((NP, 1), jnp.float32),
            jax.ShapeDtypeStruct((1, 16), jnp.float32),
        ],
    )(x, W, att_src.reshape(1, H), att_dst.reshape(1, H))


def _edge_body(h_hbm, as_hbm, ad_hbm, sh_hbm, src_hbm, dst_hbm,
               out_hbm, den_hbm,
               shift_v, idxb, avdv, wbuf, rows_g, out_sp, den_sp,
               isem0, isem1, isem2, gsem0, gsem1, gsem2,
               ssem0, ssem1, ssem2, asem0, asem1, asem2,
               wsem0, wsem1, wsem2):
    cid = lax.axis_index("c")
    sid = lax.axis_index("s")
    worker = cid * NS + sid
    isems = (isem0, isem1, isem2)
    gsems = (gsem0, gsem1, gsem2)
    ssems = (ssem0, ssem1, ssem2)
    asems = (asem0, asem1, asem2)
    wsems = (wsem0, wsem1, wsem2)

    ebase = worker * (K * C)

    def idx_copy_s(k, s, sem):
        return pltpu.make_async_copy(src_hbm.at[pl.ds(ebase + k * C, C)],
                                     idxb.at[s, 0], sem)

    def idx_copy_d(k, s, sem):
        return pltpu.make_async_copy(dst_hbm.at[pl.ds(ebase + k * C, C)],
                                     idxb.at[s, 1], sem)

    def row_gather(s, sem):
        return pltpu.make_async_copy(h_hbm.at[idxb.at[s, 0]], rows_g.at[s],
                                     sem)

    def logit_gather_a(s, sem):
        return pltpu.make_async_copy(as_hbm.at[idxb.at[s, 0]],
                                     avdv.at[s, 0], sem)

    def logit_gather_b(s, sem):
        return pltpu.make_async_copy(ad_hbm.at[idxb.at[s, 1]],
                                     avdv.at[s, 1], sem)

    def row_scatter(s, sem):
        return pltpu.make_async_copy(rows_g.at[s], out_sp.at[idxb.at[s, 1]],
                                     sem)

    def den_scatter(s, sem):
        return pltpu.make_async_copy(wbuf.at[s], den_sp.at[idxb.at[s, 1]],
                                     sem)

    pltpu.sync_copy(sh_hbm, shift_v)
    shift_vec = shift_v[...]

    zf = jnp.zeros((16,), jnp.float32)

    @pl.loop(0, C)
    def _(r):
        for t in range(H // 16):
            rows_g[0, r, pl.ds(16 * t, 16)] = zf

    for t in range(C // 16):
        wbuf[0, pl.ds(16 * t, 16)] = zf

    row0 = sid * ROWS_PER_TILE
    for off, sz in _ZCHUNKS:
        pltpu.sync_copy(rows_g.at[0, pl.ds(0, sz)],
                        out_sp.at[pl.ds(row0 + off, sz)])
        pltpu.sync_copy(wbuf.at[0, pl.ds(0, sz)],
                        den_sp.at[pl.ds(row0 + off, sz)])
    plsc.subcore_barrier()

    idx_copy_s(0, 0, isem0).start()
    idx_copy_d(0, 0, isem0).start()
    idx_copy_s(1, 1, isem1).start()
    idx_copy_d(1, 1, isem1).start()
    idx_copy_s(0, 0, isem0).wait()
    idx_copy_d(0, 0, isem0).wait()
    row_gather(0, gsem0).start()
    logit_gather_a(0, asem0).start()
    logit_gather_b(0, asem0).start()

    @pl.loop(0, K, step=3)
    def _(k0):
        for b in range(3):
            j = k0 + b
            s = b
            s1 = (b + 1) % 3
            sm1 = (b + 2) % 3
            logit_gather_a(s, asems[s]).wait()
            logit_gather_b(s, asems[s]).wait()
            for t in range(C // 16):
                sl = pl.ds(16 * t, 16)
                e = avdv[s, 0, sl] + avdv[s, 1, sl]
                e = jnp.where(e < 0, e * jnp.float32(0.2), e)
                wbuf[s, sl] = jnp.exp(e - shift_vec)

            @pl.when(j + 1 < K)
            def _():
                idx_copy_s(j + 1, s1, isems[s1]).wait()
                idx_copy_d(j + 1, s1, isems[s1]).wait()
                row_gather(s1, gsems[s1]).start()
                logit_gather_a(s1, asems[s1]).start()
                logit_gather_b(s1, asems[s1]).start()

            row_gather(s, gsems[s]).wait()

            @plsc.parallel_loop(0, C, unroll=8)
            def _(r):
                wv = plsc.load_gather(wbuf.at[s],
                                      [jnp.full((16,), r, jnp.int32)])
                for t in range(H // 16):
                    sl = pl.ds(16 * t, 16)
                    rows_g[s, r, sl] = rows_g[s, r, sl] * wv

            row_scatter(s, ssems[s]).start(add=True)
            den_scatter(s, wsems[s]).start(add=True)

            @pl.when(j >= 1)
            def _():
                @pl.when(j + 2 < K)
                def _():
                    row_scatter(sm1, ssems[sm1]).wait()
                    den_scatter(sm1, wsems[sm1]).wait()
                    idx_copy_s(j + 2, sm1, isems[sm1]).start()
                    idx_copy_d(j + 2, sm1, isems[sm1]).start()

            @pl.when(j == 0)
            def _():
                idx_copy_s(2, 2, isem2).start()
                idx_copy_d(2, 2, isem2).start()

    for s in range(3):
        row_scatter(s, ssems[s]).wait()
        den_scatter(s, wsems[s]).wait()
    plsc.subcore_barrier()
    for off, sz in _OCHUNKS:
        pltpu.sync_copy(out_sp.at[pl.ds(row0 + off, sz)],
                        out_hbm.at[cid, pl.ds(row0 + off, sz)])
        pltpu.sync_copy(den_sp.at[pl.ds(row0 + off, sz)],
                        den_hbm.at[pl.ds(cid * NP + row0 + off, sz)])


def _edge_pass(h_pad, a_src, a_dst, shift, src_1d, dst_1d):
    mesh = plsc.VectorSubcoreMesh(core_axis_name="c", subcore_axis_name="s")
    kern = pl.kernel(
        _edge_body,
        out_type=[
            jax.ShapeDtypeStruct((NC, NP, H), jnp.float32),
            jax.ShapeDtypeStruct((NC * NP,), jnp.float32),
        ],
        mesh=mesh,
        compiler_params=pltpu.CompilerParams(needs_layout_passes=False),
        scratch_types=[
            pltpu.VMEM((16,), jnp.float32),
            pltpu.VMEM((3, 2, C), jnp.int32),
            pltpu.VMEM((3, 2, C), jnp.float32),
            pltpu.VMEM((3, C), jnp.float32),
            pltpu.VMEM((3, C, H), jnp.float32),
            pltpu.VMEM_SHARED((NP, H), jnp.float32),
            pltpu.VMEM_SHARED((NP,), jnp.float32),
        ] + [pltpu.SemaphoreType.DMA] * 15,
    )
    return kern(h_pad, a_src, a_dst, shift, src_1d, dst_1d)


def _final_body(p_ref, d_ref, h_ref, as_ref, ad_ref, sh_ref,
                b_ref, g_ref, be_ref, m_ref, v_ref, o_ref):
    e = as_ref[...] + ad_ref[...]
    e = jnp.where(e < 0, e * 0.2, e)
    w_self = jnp.exp(e - sh_ref[0, 0])
    num = p_ref[0] + p_ref[1] + w_self * h_ref[...]
    den = d_ref[0] + d_ref[1] + w_self + 1e-16
    out = num / den + b_ref[...]
    out = jnp.maximum(out, 0.0)
    scale = g_ref[...] * jax.lax.rsqrt(v_ref[...] + 1e-5)
    o_ref[...] = (out - m_ref[...]) * scale + be_ref[...]


def _finalize(partials, denoms, h_pad, a_src, a_dst, shift,
              bias, bn_gamma, bn_beta, bn_mean, bn_var):
    blk = 1000
    vec = lambda a: a.reshape(1, H)
    full = lambda i: (0, 0)
    return pl.pallas_call(
        _final_body,
        grid=(N // blk,),
        in_specs=[
            pl.BlockSpec((NC, blk, H), lambda i: (0, i, 0)),
            pl.BlockSpec((NC, blk, 1), lambda i: (0, i, 0)),
            pl.BlockSpec((blk, H), lambda i: (i, 0)),
            pl.BlockSpec((blk, 1), lambda i: (i, 0)),
            pl.BlockSpec((blk, 1), lambda i: (i, 0)),
            pl.BlockSpec((1, 16), full),
            pl.BlockSpec((1, H), full),
            pl.BlockSpec((1, H), full),
            pl.BlockSpec((1, H), full),
            pl.BlockSpec((1, H), full),
            pl.BlockSpec((1, H), full),
        ],
        out_specs=pl.BlockSpec((blk, H), lambda i: (i, 0)),
        out_shape=jax.ShapeDtypeStruct((N, H), jnp.float32),
    )(partials, denoms.reshape(NC, NP, 1), h_pad, a_src, a_dst, shift,
      vec(bias), vec(bn_gamma), vec(bn_beta), vec(bn_mean), vec(bn_var))


def kernel(x, edge_index, W, att_src, att_dst, bias, bn_gamma, bn_beta,
           bn_mean, bn_var):
    pad = N + (jnp.arange(E_PAD - E_RAW, dtype=jnp.int32) % (NP - N))
    src_1d = jnp.concatenate([edge_index[0], pad])
    dst_1d = jnp.concatenate([edge_index[1], pad])

    h_pad, a_src, a_dst, shift = _project(x, W, att_src, att_dst)
    partials, denoms = _edge_pass(
        h_pad, a_src.reshape(NP), a_dst.reshape(NP), shift.reshape(16),
        src_1d, dst_1d)
    return _finalize(partials, denoms, h_pad, a_src, a_dst, shift,
                     bias, bn_gamma, bn_beta, bn_mean, bn_var)


# speedup vs baseline: 2.4165x; 1.1283x over previous
import functools

import jax
import jax.numpy as jnp
from jax import lax
from jax.experimental import pallas as pl
from jax.experimental.pallas import tpu as pltpu
from jax.experimental.pallas import tpu_sc as plsc

N = 10000
NP = 10240
H = 128
E_RAW = 320000
NC = 2
NS = 16
NW = NC * NS
C = 112
K = 90
E_PAD = NW * C * K
ROWS_PER_TILE = NP // NS
_ZCHUNKS = [(i * C, min(C, ROWS_PER_TILE - i * C))
            for i in range(-(-ROWS_PER_TILE // C))]
_OCHUNKS = [(i * 128, 128) for i in range(ROWS_PER_TILE // 128)]


def _proj_body(x_ref, w_ref, asv_ref, adv_ref, h_ref, as_ref, ad_ref, sh_ref):
    x = x_ref[...]
    w = w_ref[...]
    h = jnp.dot(x, w, preferred_element_type=jnp.float32)
    h_ref[pl.ds(0, N), :] = h
    h_ref[pl.ds(N, NP - N), :] = jnp.zeros((NP - N, H), jnp.float32)
    dn = (((1,), (1,)), ((), ()))
    u_s = lax.dot_general(asv_ref[...], w, dn,
                          preferred_element_type=jnp.float32)
    u_d = lax.dot_general(adv_ref[...], w, dn,
                          preferred_element_type=jnp.float32)
    a_sT = lax.dot_general(u_s, x, dn, preferred_element_type=jnp.float32)
    a_dT = lax.dot_general(u_d, x, dn, preferred_element_type=jnp.float32)
    as_ref[:, pl.ds(0, N)] = a_sT
    ad_ref[:, pl.ds(0, N)] = a_dT
    pad = jnp.full((1, NP - N), -1e30, jnp.float32)
    as_ref[:, pl.ds(N, NP - N)] = pad
    ad_ref[:, pl.ds(N, NP - N)] = pad
    shift = jnp.max(a_sT) + jnp.max(a_dT)
    sh_ref[...] = jnp.full((1, 16), shift, jnp.float32)


def _project(x, W, att_src, att_dst):
    return pl.pallas_call(
        _proj_body,
        out_shape=[
            jax.ShapeDtypeStruct((NP, H), jnp.float32),
            jax.ShapeDtypeStruct((1, NP), jnp.float32),
            jax.ShapeDtypeStruct((1, NP), jnp.float32),
            jax.ShapeDtypeStruct((1, 16), jnp.float32),
        ],
    )(x, W, att_src.reshape(1, H), att_dst.reshape(1, H))


def _edge_body(h_hbm, as_hbm, ad_hbm, sh_hbm, src_hbm, dst_hbm,
               out_hbm, den_hbm,
               shift_v, idxb, avdv, wbuf, rows_g, out_sp, den_sp,
               isem0, isem1, isem2, gsem0, gsem1, gsem2,
               ssem0, ssem1, ssem2, asem0, asem1, asem2,
               wsem0, wsem1, wsem2):
    cid = lax.axis_index("c")
    sid = lax.axis_index("s")
    worker = cid * NS + sid
    isems = (isem0, isem1, isem2)
    gsems = (gsem0, gsem1, gsem2)
    ssems = (ssem0, ssem1, ssem2)
    asems = (asem0, asem1, asem2)
    wsems = (wsem0, wsem1, wsem2)

    ebase = worker * (K * C)

    def idx_copy_s(k, s, sem):
        return pltpu.make_async_copy(src_hbm.at[pl.ds(ebase + k * C, C)],
                                     idxb.at[s, 0], sem)

    def idx_copy_d(k, s, sem):
        return pltpu.make_async_copy(dst_hbm.at[pl.ds(ebase + k * C, C)],
                                     idxb.at[s, 1], sem)

    def row_gather(s, sem):
        return pltpu.make_async_copy(h_hbm.at[idxb.at[s, 0]], rows_g.at[s],
                                     sem)

    def logit_gather_a(s, sem):
        return pltpu.make_async_copy(as_hbm.at[idxb.at[s, 0]],
                                     avdv.at[s, 0], sem)

    def logit_gather_b(s, sem):
        return pltpu.make_async_copy(ad_hbm.at[idxb.at[s, 1]],
                                     avdv.at[s, 1], sem)

    def row_scatter(s, sem):
        return pltpu.make_async_copy(rows_g.at[s], out_sp.at[idxb.at[s, 1]],
                                     sem)

    def den_scatter(s, sem):
        return pltpu.make_async_copy(wbuf.at[s], den_sp.at[idxb.at[s, 1]],
                                     sem)

    pltpu.sync_copy(sh_hbm, shift_v)
    shift_vec = shift_v[...]

    zf = jnp.zeros((16,), jnp.float32)

    @pl.loop(0, C)
    def _(r):
        for t in range(H // 16):
            rows_g[0, r, pl.ds(16 * t, 16)] = zf

    for t in range(C // 16):
        wbuf[0, pl.ds(16 * t, 16)] = zf

    row0 = sid * ROWS_PER_TILE
    for off, sz in _ZCHUNKS:
        pltpu.sync_copy(rows_g.at[0, pl.ds(0, sz)],
                        out_sp.at[pl.ds(row0 + off, sz)])
        pltpu.sync_copy(wbuf.at[0, pl.ds(0, sz)],
                        den_sp.at[pl.ds(row0 + off, sz)])
    plsc.subcore_barrier()

    idx_copy_s(0, 0, isem0).start()
    idx_copy_d(0, 0, isem0).start()
    idx_copy_s(1, 1, isem1).start()
    idx_copy_d(1, 1, isem1).start()
    idx_copy_s(0, 0, isem0).wait()
    idx_copy_d(0, 0, isem0).wait()
    row_gather(0, gsem0).start()
    logit_gather_a(0, asem0).start()
    logit_gather_b(0, asem0).start()

    @pl.loop(0, K, step=3)
    def _(k0):
        for b in range(3):
            j = k0 + b
            s = b
            s1 = (b + 1) % 3
            sm1 = (b + 2) % 3
            logit_gather_a(s, asems[s]).wait()
            logit_gather_b(s, asems[s]).wait()
            for t in range(C // 16):
                sl = pl.ds(16 * t, 16)
                e = avdv[s, 0, sl] + avdv[s, 1, sl]
                e = jnp.where(e < 0, e * jnp.float32(0.2), e)
                wbuf[s, sl] = jnp.exp(e - shift_vec)

            @pl.when(j + 1 < K)
            def _():
                idx_copy_s(j + 1, s1, isems[s1]).wait()
                idx_copy_d(j + 1, s1, isems[s1]).wait()
                row_gather(s1, gsems[s1]).start()
                logit_gather_a(s1, asems[s1]).start()
                logit_gather_b(s1, asems[s1]).start()

            row_gather(s, gsems[s]).wait()

            @plsc.parallel_loop(0, C, unroll=8)
            def _(r):
                wv = plsc.load_gather(wbuf.at[s],
                                      [jnp.full((16,), r, jnp.int32)])
                for t in range(H // 16):
                    sl = pl.ds(16 * t, 16)
                    rows_g[s, r, sl] = rows_g[s, r, sl] * wv

            row_scatter(s, ssems[s]).start(add=True)
            den_scatter(s, wsems[s]).start(add=True)

            @pl.when(j >= 1)
            def _():
                @pl.when(j + 2 < K)
                def _():
                    row_scatter(sm1, ssems[sm1]).wait()
                    den_scatter(sm1, wsems[sm1]).wait()
                    idx_copy_s(j + 2, sm1, isems[sm1]).start()
                    idx_copy_d(j + 2, sm1, isems[sm1]).start()

            @pl.when(j == 0)
            def _():
                idx_copy_s(2, 2, isem2).start()
                idx_copy_d(2, 2, isem2).start()

    for s in range(3):
        row_scatter(s, ssems[s]).wait()
        den_scatter(s, wsems[s]).wait()
    plsc.subcore_barrier()
    for off, sz in _OCHUNKS:
        pltpu.sync_copy(out_sp.at[pl.ds(row0 + off, sz)],
                        out_hbm.at[cid, pl.ds(row0 + off, sz)])
        pltpu.sync_copy(den_sp.at[pl.ds(row0 + off, sz)],
                        den_hbm.at[pl.ds(cid * NP + row0 + off, sz)])


def _edge_pass(h_pad, a_src, a_dst, shift, src_1d, dst_1d):
    mesh = plsc.VectorSubcoreMesh(core_axis_name="c", subcore_axis_name="s")
    kern = pl.kernel(
        _edge_body,
        out_type=[
            jax.ShapeDtypeStruct((NC, NP, H), jnp.float32),
            jax.ShapeDtypeStruct((NC * NP,), jnp.float32),
        ],
        mesh=mesh,
        compiler_params=pltpu.CompilerParams(needs_layout_passes=False),
        scratch_types=[
            pltpu.VMEM((16,), jnp.float32),
            pltpu.VMEM((3, 2, C), jnp.int32),
            pltpu.VMEM((3, 2, C), jnp.float32),
            pltpu.VMEM((3, C), jnp.float32),
            pltpu.VMEM((3, C, H), jnp.float32),
            pltpu.VMEM_SHARED((NP, H), jnp.float32),
            pltpu.VMEM_SHARED((NP,), jnp.float32),
        ] + [pltpu.SemaphoreType.DMA] * 15,
    )
    return kern(h_pad, a_src, a_dst, shift, src_1d, dst_1d)


_FBLK = 1024


def _final_body(p_ref, d_ref, h_ref, sh_ref, asv_ref, adv_ref,
                b_ref, g_ref, be_ref, m_ref, v_ref, o_ref):
    h = h_ref[...]
    a_s = jnp.sum(h * asv_ref[...], axis=1, keepdims=True)
    a_d = jnp.sum(h * adv_ref[...], axis=1, keepdims=True)
    e = a_s + a_d
    e = jnp.where(e < 0, e * 0.2, e)
    w_self = jnp.exp(e - sh_ref[0, 0])
    num = p_ref[0] + p_ref[1] + w_self * h
    d3 = lax.broadcast_in_dim(d_ref[0] + d_ref[1], (_FBLK // 128, 128, 128),
                              (0, 2))
    d_exp = jnp.reshape(d3, (_FBLK, 128))
    lane = lax.broadcasted_iota(jnp.int32, (_FBLK, 128), 1)
    rowm = lax.broadcasted_iota(jnp.int32, (_FBLK, 128), 0) % 128
    den_col = jnp.sum(jnp.where(lane == rowm, d_exp, 0.0), axis=1,
                      keepdims=True)
    den = den_col + w_self + 1e-16
    out = num / den + b_ref[...]
    out = jnp.maximum(out, 0.0)
    scale = g_ref[...] * jax.lax.rsqrt(v_ref[...] + 1e-5)
    o_ref[...] = (out - m_ref[...]) * scale + be_ref[...]


def _finalize(partials, denoms, h_pad, shift,
              att_src, att_dst, bias, bn_gamma, bn_beta, bn_mean, bn_var):
    blk = _FBLK
    vec = lambda a: a.reshape(1, H)
    full = lambda i: (0, 0)
    out = pl.pallas_call(
        _final_body,
        grid=(NP // blk,),
        in_specs=[
            pl.BlockSpec((NC, blk, H), lambda i: (0, i, 0)),
            pl.BlockSpec((NC, blk // 128, 128), lambda i: (0, i, 0)),
            pl.BlockSpec((blk, H), lambda i: (i, 0)),
            pl.BlockSpec((1, 16), full),
            pl.BlockSpec((1, H), full),
            pl.BlockSpec((1, H), full),
            pl.BlockSpec((1, H), full),
            pl.BlockSpec((1, H), full),
            pl.BlockSpec((1, H), full),
            pl.BlockSpec((1, H), full),
            pl.BlockSpec((1, H), full),
        ],
        out_specs=pl.BlockSpec((blk, H), lambda i: (i, 0)),
        out_shape=jax.ShapeDtypeStruct((NP, H), jnp.float32),
    )(partials, denoms.reshape(NC, NP // 128, 128), h_pad, shift,
      vec(att_src), vec(att_dst), vec(bias), vec(bn_gamma), vec(bn_beta),
      vec(bn_mean), vec(bn_var))
    return out[:N]


def kernel(x, edge_index, W, att_src, att_dst, bias, bn_gamma, bn_beta,
           bn_mean, bn_var):
    pad = N + (jnp.arange(E_PAD - E_RAW, dtype=jnp.int32) % (NP - N))
    src_1d = jnp.concatenate([edge_index[0], pad])
    dst_1d = jnp.concatenate([edge_index[1], pad])

    h_pad, a_src, a_dst, shift = _project(x, W, att_src, att_dst)
    partials, denoms = _edge_pass(
        h_pad, a_src.reshape(NP), a_dst.reshape(NP), shift.reshape(16),
        src_1d, dst_1d)
    return _finalize(partials, denoms, h_pad, shift, att_src, att_dst,
                     bias, bn_gamma, bn_beta, bn_mean, bn_var)
